# baseline (speedup 1.0000x reference)
import functools

import jax
import jax.numpy as jnp
from jax import lax
from jax.experimental import pallas as pl
from jax.experimental.pallas import tpu as pltpu
from jax.experimental.pallas import tpu_sc as plsc


def kernel(inputs, weight):
    *batches, d1, d2 = inputs.shape
    nbatch = 1
    for b in batches:
        nbatch *= b

    info = plsc.get_sparse_core_info()
    nworkers = info.num_cores * info.num_subcores
    rows_per_worker = d1 // nworkers
    chunk_rows = min(rows_per_worker, 64)
    nchunks = rows_per_worker // chunk_rows

    mesh = plsc.VectorSubcoreMesh(core_axis_name="c", subcore_axis_name="s")

    @functools.partial(
        pl.kernel,
        mesh=mesh,
        out_type=jax.ShapeDtypeStruct((nbatch, d1, d2), weight.dtype),
        scratch_types=[
            pltpu.VMEM((chunk_rows, d2), weight.dtype),
            pltpu.SemaphoreType.DMA,
        ],
    )
    def sc_copy(w_hbm, o_hbm, buf, sem):
        wid = lax.axis_index("s") * info.num_cores + lax.axis_index("c")
        base = wid * rows_per_worker

        def chunk_body(c, carry):
            row0 = base + c * chunk_rows
            pltpu.sync_copy(w_hbm.at[pl.ds(row0, chunk_rows), :], buf)
            copies = [
                pltpu.make_async_copy(
                    buf, o_hbm.at[b, pl.ds(row0, chunk_rows), :], sem
                )
                for b in range(nbatch)
            ]
            for cp in copies:
                cp.start()
            for cp in copies:
                cp.wait()
            return carry

        lax.fori_loop(0, nchunks, chunk_body, 0)

    out = sc_copy(weight)
    return out.reshape(tuple(batches) + (d1, d2))


# speedup vs baseline: 1.0040x; 1.0040x over previous
import functools

import jax
import jax.numpy as jnp
from jax import lax
from jax.experimental import pallas as pl
from jax.experimental.pallas import tpu as pltpu
from jax.experimental.pallas import tpu_sc as plsc


def kernel(inputs, weight):
    *batches, d1, d2 = inputs.shape
    nbatch = 1
    for b in batches:
        nbatch *= b

    info = plsc.get_sparse_core_info()
    nworkers = info.num_cores * info.num_subcores
    rows_per_worker = d1 // nworkers
    chunk_rows = min(rows_per_worker, 32)
    nchunks = rows_per_worker // chunk_rows

    mesh = plsc.VectorSubcoreMesh(core_axis_name="c", subcore_axis_name="s")

    @functools.partial(
        pl.kernel,
        mesh=mesh,
        out_type=jax.ShapeDtypeStruct((nbatch, d1, d2), weight.dtype),
        scratch_types=[
            pltpu.VMEM((2, chunk_rows, d2), weight.dtype),
            pltpu.SemaphoreType.DMA,
            pltpu.SemaphoreType.DMA,
        ],
    )
    def sc_copy(w_hbm, o_hbm, buf, sem_in, sem_out):
        wid = lax.axis_index("s") * info.num_cores + lax.axis_index("c")
        base = wid * rows_per_worker

        def in_copy(c):
            row0 = base + c * chunk_rows
            return pltpu.make_async_copy(
                w_hbm.at[pl.ds(row0, chunk_rows), :], buf.at[c % 2], sem_in
            )

        def out_copies(c):
            row0 = base + c * chunk_rows
            return [
                pltpu.make_async_copy(
                    buf.at[c % 2], o_hbm.at[b, pl.ds(row0, chunk_rows), :], sem_out
                )
                for b in range(nbatch)
            ]

        in_copy(0).start()
        for c in range(nchunks):
            if c >= 1:
                for cp in out_copies(c - 1):
                    cp.wait()
            if c + 1 < nchunks:
                in_copy(c + 1).start()
            in_copy(c).wait()
            for cp in out_copies(c):
                cp.start()
        for cp in out_copies(nchunks - 1):
            cp.wait()

    out = sc_copy(weight)
    return out.reshape(tuple(batches) + (d1, d2))
